# initial kernel scaffold (unmeasured)
import jax
import jax.numpy as jnp
from jax import lax
from jax.experimental import pallas as pl
from jax.experimental.pallas import tpu as pltpu

N_DEV = 8
HL = 4
DH = 64
NEG = -1e9


def kernel(x, Wq, K_ext, V_ext, Wo):
    B, Sq, Dm = x.shape
    Skv_l = K_ext.shape[1]
    bf16 = jnp.bfloat16

    x16 = x.astype(bf16)
    wq16 = Wq.astype(bf16)
    wo16 = Wo.astype(bf16)
    k_t = jnp.transpose(K_ext, (2, 0, 1, 3)).astype(bf16)
    v_t = jnp.transpose(V_ext, (2, 0, 1, 3)).astype(bf16)

    def body(x_ref, wq_ref, k_ref, v_ref, wo_ref, out_ref,
             q_buf, k_gath, v_gath, scores, ctx_buf, out_gath,
             send_k, recv_k, send_v, recv_v, send_o, recv_o):
        my = lax.axis_index("i")

        barrier = pltpu.get_barrier_semaphore()
        for d in range(1, N_DEV):
            pl.semaphore_signal(barrier, inc=1,
                                device_id=((my + d) % N_DEV,),
                                device_id_type=pl.DeviceIdType.MESH)
        pl.semaphore_wait(barrier, N_DEV - 1)

        def kv_rdma(d, p, slot):
            rk = pltpu.make_async_remote_copy(
                src_ref=k_ref.at[pl.ds(HL * p, HL)], dst_ref=k_gath.at[slot],
                send_sem=send_k.at[d], recv_sem=recv_k.at[slot],
                device_id=(p,), device_id_type=pl.DeviceIdType.MESH)
            rv = pltpu.make_async_remote_copy(
                src_ref=v_ref.at[pl.ds(HL * p, HL)], dst_ref=v_gath.at[slot],
                send_sem=send_v.at[d], recv_sem=recv_v.at[slot],
                device_id=(p,), device_id_type=pl.DeviceIdType.MESH)
            return rk, rv

        def o_rdma(d, p, src_slot, dst_slot):
            return pltpu.make_async_remote_copy(
                src_ref=out_gath.at[src_slot], dst_ref=out_gath.at[dst_slot],
                send_sem=send_o.at[d], recv_sem=recv_o.at[dst_slot],
                device_id=(p,), device_id_type=pl.DeviceIdType.MESH)

        for d in range(1, N_DEV):
            p = (my + d) % N_DEV
            rk, rv = kv_rdma(d, p, my)
            rk.start()
            rv.start()

        k_gath[my] = k_ref[pl.ds(HL * my, HL)]
        v_gath[my] = v_ref[pl.ds(HL * my, HL)]

        for b in range(B):
            q = lax.dot_general(x_ref[b], wq_ref[...],
                                (((1,), (0,)), ((), ())),
                                preferred_element_type=jnp.float32)
            q_buf[b] = (q * 0.125).astype(bf16)

        for d in range(1, N_DEV):
            s = (my - d) % N_DEV
            rk, rv = kv_rdma(d, s, s)
            rk.wait_recv()
            rv.wait_recv()

        for b in range(B):
            for h in range(HL):
                qbh = q_buf[b, :, h * DH:(h + 1) * DH]
                for c in range(N_DEV):
                    kc = k_gath[c, h, b]
                    blk = lax.dot_general(qbh, kc, (((1,), (1,)), ((), ())),
                                          preferred_element_type=jnp.float32)
                    qi = lax.broadcasted_iota(jnp.int32, (Sq, Skv_l), 0)
                    kj = (lax.broadcasted_iota(jnp.int32, (Sq, Skv_l), 1)
                          + c * Skv_l)
                    msk = (jnp.abs(qi - kj) <= 128) | (kj < 32) | (qi < 32)
                    scores[:, c * Skv_l:(c + 1) * Skv_l] = (
                        jnp.where(msk, blk, NEG))
                sc = scores[...]
                mx = jnp.max(sc, axis=1, keepdims=True)
                w = jnp.exp(sc - mx)
                denom = jnp.sum(sc - mx, axis=1, keepdims=True)
                denom = jnp.sum(w, axis=1, keepdims=True)
                w16 = w.astype(bf16)
                ctx = lax.dot_general(
                    w16[:, 0:Skv_l], v_gath[0, h, b],
                    (((1,), (0,)), ((), ())),
                    preferred_element_type=jnp.float32)
                for c in range(1, N_DEV):
                    ctx = ctx + lax.dot_general(
                        w16[:, c * Skv_l:(c + 1) * Skv_l], v_gath[c, h, b],
                        (((1,), (0,)), ((), ())),
                        preferred_element_type=jnp.float32)
                ctx = ctx / denom
                ctx_buf[b, :, h * DH:(h + 1) * DH] = ctx.astype(bf16)

        po = jnp.stack([
            lax.dot_general(ctx_buf[b], wo_ref[...], (((1,), (0,)), ((), ())),
                            preferred_element_type=jnp.float32)
            for b in range(B)])
        out_gath[my] = po.astype(bf16)

        for d in range(1, N_DEV):
            p = (my + d) % N_DEV
            o_rdma(d, p, my, my).start()
        for d in range(1, N_DEV):
            s = (my - d) % N_DEV
            o_rdma(d, s, s, s).wait_recv()

        for b in range(B):
            acc = out_gath[0, b].astype(jnp.float32)
            for c in range(1, N_DEV):
                acc = acc + out_gath[c, b].astype(jnp.float32)
            out_ref[b] = acc

        for d in range(1, N_DEV):
            p = (my + d) % N_DEV
            rk, rv = kv_rdma(d, p, my)
            rk.wait_send()
            rv.wait_send()
            o_rdma(d, p, my, my).wait_send()

    return pl.pallas_call(
        body,
        out_shape=jax.ShapeDtypeStruct((B, Sq, Dm), jnp.float32),
        in_specs=[pl.BlockSpec(memory_space=pltpu.VMEM)] * 5,
        out_specs=pl.BlockSpec(memory_space=pltpu.VMEM),
        scratch_shapes=[
            pltpu.VMEM((B, Sq, HL * DH), bf16),
            pltpu.VMEM((N_DEV, HL, B, Skv_l, DH), bf16),
            pltpu.VMEM((N_DEV, HL, B, Skv_l, DH), bf16),
            pltpu.VMEM((Sq, N_DEV * Skv_l), jnp.float32),
            pltpu.VMEM((B, Sq, HL * DH), bf16),
            pltpu.VMEM((N_DEV, B, Sq, Dm), bf16),
            pltpu.SemaphoreType.DMA((N_DEV,)),
            pltpu.SemaphoreType.DMA((N_DEV,)),
            pltpu.SemaphoreType.DMA((N_DEV,)),
            pltpu.SemaphoreType.DMA((N_DEV,)),
            pltpu.SemaphoreType.DMA((N_DEV,)),
            pltpu.SemaphoreType.DMA((N_DEV,)),
        ],
        compiler_params=pltpu.CompilerParams(collective_id=0),
    )(x16, wq16, k_t, v_t, wo16)


# baseline (device time: 111249 ns/iter reference)
import jax
import jax.numpy as jnp
from jax import lax
from jax.experimental import pallas as pl
from jax.experimental.pallas import tpu as pltpu

N_DEV = 8
HL = 4
DH = 64
NEG = -1e9


def kernel(x, Wq, K_ext, V_ext, Wo):
    B, Sq, Dm = x.shape
    Skv_l = K_ext.shape[1]
    bf16 = jnp.bfloat16

    x16 = x.astype(bf16)
    wq16 = Wq.astype(bf16)
    wo16 = Wo.astype(bf16)
    k_t = jnp.transpose(K_ext, (2, 0, 1, 3)).astype(bf16)
    v_t = jnp.transpose(V_ext, (2, 0, 1, 3)).astype(bf16)

    def body(x_ref, wq_ref, k_ref, v_ref, wo_ref, out_ref,
             q_buf, k_gath, v_gath, scores, ctx_buf, out_gath,
             send_k, recv_k, send_v, recv_v, send_o, recv_o):
        my = lax.axis_index("i")

        barrier = pltpu.get_barrier_semaphore()
        for d in range(1, N_DEV):
            pl.semaphore_signal(barrier, inc=1,
                                device_id=((my + d) % N_DEV,),
                                device_id_type=pl.DeviceIdType.MESH)
        pl.semaphore_wait(barrier, N_DEV - 1)

        def kv_rdma(d, p, slot):
            rk = pltpu.make_async_remote_copy(
                src_ref=k_ref.at[pl.ds(HL * p, HL)], dst_ref=k_gath.at[slot],
                send_sem=send_k.at[d], recv_sem=recv_k.at[slot],
                device_id=(p,), device_id_type=pl.DeviceIdType.MESH)
            rv = pltpu.make_async_remote_copy(
                src_ref=v_ref.at[pl.ds(HL * p, HL)], dst_ref=v_gath.at[slot],
                send_sem=send_v.at[d], recv_sem=recv_v.at[slot],
                device_id=(p,), device_id_type=pl.DeviceIdType.MESH)
            return rk, rv

        def o_rdma(d, p, src_slot, dst_slot):
            return pltpu.make_async_remote_copy(
                src_ref=out_gath.at[src_slot], dst_ref=out_gath.at[dst_slot],
                send_sem=send_o.at[d], recv_sem=recv_o.at[dst_slot],
                device_id=(p,), device_id_type=pl.DeviceIdType.MESH)

        for d in range(1, N_DEV):
            p = (my + d) % N_DEV
            rk, rv = kv_rdma(d, p, my)
            rk.start()
            rv.start()

        k_gath[my] = k_ref[pl.ds(HL * my, HL)]
        v_gath[my] = v_ref[pl.ds(HL * my, HL)]

        for b in range(B):
            q = lax.dot_general(x_ref[b], wq_ref[...],
                                (((1,), (0,)), ((), ())),
                                preferred_element_type=jnp.float32)
            q_buf[b] = (q * 0.125).astype(bf16)

        for d in range(1, N_DEV):
            s = (my - d) % N_DEV
            rk, rv = kv_rdma(d, s, s)
            rk.wait_recv()
            rv.wait_recv()

        for b in range(B):
            for h in range(HL):
                qbh = q_buf[b, :, h * DH:(h + 1) * DH]
                for c in range(N_DEV):
                    kc = k_gath[c, h, b]
                    blk = lax.dot_general(qbh, kc, (((1,), (1,)), ((), ())),
                                          preferred_element_type=jnp.float32)
                    qi = lax.broadcasted_iota(jnp.int32, (Sq, Skv_l), 0)
                    kj = (lax.broadcasted_iota(jnp.int32, (Sq, Skv_l), 1)
                          + c * Skv_l)
                    msk = (jnp.abs(qi - kj) <= 128) | (kj < 32) | (qi < 32)
                    scores[:, c * Skv_l:(c + 1) * Skv_l] = (
                        jnp.where(msk, blk, NEG))
                sc = scores[...]
                mx = jnp.max(sc, axis=1, keepdims=True)
                w = jnp.exp(sc - mx)
                denom = jnp.sum(w, axis=1, keepdims=True)
                w16 = w.astype(bf16)
                ctx = lax.dot_general(
                    w16[:, 0:Skv_l], v_gath[0, h, b],
                    (((1,), (0,)), ((), ())),
                    preferred_element_type=jnp.float32)
                for c in range(1, N_DEV):
                    ctx = ctx + lax.dot_general(
                        w16[:, c * Skv_l:(c + 1) * Skv_l], v_gath[c, h, b],
                        (((1,), (0,)), ((), ())),
                        preferred_element_type=jnp.float32)
                ctx = ctx / denom
                ctx_buf[b, :, h * DH:(h + 1) * DH] = ctx.astype(bf16)

        po = jnp.stack([
            lax.dot_general(ctx_buf[b], wo_ref[...], (((1,), (0,)), ((), ())),
                            preferred_element_type=jnp.float32)
            for b in range(B)])
        out_gath[my] = po.astype(bf16)

        for d in range(1, N_DEV):
            p = (my + d) % N_DEV
            o_rdma(d, p, my, my).start()
        for d in range(1, N_DEV):
            s = (my - d) % N_DEV
            o_rdma(d, s, s, s).wait_recv()

        for b in range(B):
            acc = out_gath[0, b].astype(jnp.float32)
            for c in range(1, N_DEV):
                acc = acc + out_gath[c, b].astype(jnp.float32)
            out_ref[b] = acc

        for d in range(1, N_DEV):
            p = (my + d) % N_DEV
            rk, rv = kv_rdma(d, p, my)
            rk.wait_send()
            rv.wait_send()
            o_rdma(d, p, my, my).wait_send()

    return pl.pallas_call(
        body,
        out_shape=jax.ShapeDtypeStruct((B, Sq, Dm), jnp.float32),
        in_specs=[pl.BlockSpec(memory_space=pltpu.VMEM)] * 5,
        out_specs=pl.BlockSpec(memory_space=pltpu.VMEM),
        scratch_shapes=[
            pltpu.VMEM((B, Sq, HL * DH), bf16),
            pltpu.VMEM((N_DEV, HL, B, Skv_l, DH), bf16),
            pltpu.VMEM((N_DEV, HL, B, Skv_l, DH), bf16),
            pltpu.VMEM((Sq, N_DEV * Skv_l), jnp.float32),
            pltpu.VMEM((B, Sq, HL * DH), bf16),
            pltpu.VMEM((N_DEV, B, Sq, Dm), bf16),
            pltpu.SemaphoreType.DMA((N_DEV,)),
            pltpu.SemaphoreType.DMA((N_DEV,)),
            pltpu.SemaphoreType.DMA((N_DEV,)),
            pltpu.SemaphoreType.DMA((N_DEV,)),
            pltpu.SemaphoreType.DMA((N_DEV,)),
            pltpu.SemaphoreType.DMA((N_DEV,)),
        ],
        compiler_params=pltpu.CompilerParams(collective_id=0),
    )(x16, wq16, k_t, v_t, wo16)


# device time: 107701 ns/iter; 1.0329x vs baseline; 1.0329x over previous
import jax
import jax.numpy as jnp
from jax import lax
from jax.experimental import pallas as pl
from jax.experimental.pallas import tpu as pltpu

N_DEV = 8
HL = 4
DH = 64
NEG = -1e9


def kernel(x, Wq, K_ext, V_ext, Wo):
    B, Sq, Dm = x.shape
    Skv_l = K_ext.shape[1]
    bf16 = jnp.bfloat16

    x16 = x.astype(bf16)
    wq16 = Wq.astype(bf16)
    wo16 = Wo.astype(bf16)
    k_t = jnp.transpose(K_ext, (2, 0, 1, 3)).astype(bf16)
    v_t = jnp.transpose(V_ext, (2, 0, 1, 3)).astype(bf16)

    def body(x_ref, wq_ref, k_ref, v_ref, wo_ref, out_ref,
             q_buf, k_gath, v_gath, scores, ctx_buf, acc_buf, stage,
             send_k, recv_k, send_v, recv_v, send_o, recv_o):
        my = lax.axis_index("i")

        barrier = pltpu.get_barrier_semaphore()
        for d in range(1, N_DEV):
            pl.semaphore_signal(barrier, inc=1,
                                device_id=((my + d) % N_DEV,),
                                device_id_type=pl.DeviceIdType.MESH)
        pl.semaphore_wait(barrier, N_DEV - 1)

        def kv_rdma(d, p, slot):
            rk = pltpu.make_async_remote_copy(
                src_ref=k_ref.at[pl.ds(HL * p, HL)], dst_ref=k_gath.at[slot],
                send_sem=send_k.at[d], recv_sem=recv_k.at[slot],
                device_id=(p,), device_id_type=pl.DeviceIdType.MESH)
            rv = pltpu.make_async_remote_copy(
                src_ref=v_ref.at[pl.ds(HL * p, HL)], dst_ref=v_gath.at[slot],
                send_sem=send_v.at[d], recv_sem=recv_v.at[slot],
                device_id=(p,), device_id_type=pl.DeviceIdType.MESH)
            return rk, rv

        for d in range(1, N_DEV):
            p = (my + d) % N_DEV
            rk, rv = kv_rdma(d, p, my)
            rk.start()
            rv.start()

        k_gath[my] = k_ref[pl.ds(HL * my, HL)]
        v_gath[my] = v_ref[pl.ds(HL * my, HL)]

        for b in range(B):
            q = lax.dot_general(x_ref[b], wq_ref[...],
                                (((1,), (0,)), ((), ())),
                                preferred_element_type=jnp.float32)
            q_buf[b] = (q * 0.125).astype(bf16)

        for d in range(1, N_DEV):
            s = (my - d) % N_DEV
            rk, rv = kv_rdma(d, s, s)
            rk.wait_recv()
            rv.wait_recv()

        for b in range(B):
            for h in range(HL):
                qbh = q_buf[b, :, h * DH:(h + 1) * DH]
                for c in range(N_DEV):
                    kc = k_gath[c, h, b]
                    blk = lax.dot_general(qbh, kc, (((1,), (1,)), ((), ())),
                                          preferred_element_type=jnp.float32)
                    qi = lax.broadcasted_iota(jnp.int32, (Sq, Skv_l), 0)
                    kj = (lax.broadcasted_iota(jnp.int32, (Sq, Skv_l), 1)
                          + c * Skv_l)
                    msk = (jnp.abs(qi - kj) <= 128) | (kj < 32) | (qi < 32)
                    scores[:, c * Skv_l:(c + 1) * Skv_l] = (
                        jnp.where(msk, blk, NEG))
                sc = scores[...]
                mx = jnp.max(sc, axis=1, keepdims=True)
                w = jnp.exp(sc - mx)
                denom = jnp.sum(w, axis=1, keepdims=True)
                w16 = w.astype(bf16)
                ctx = lax.dot_general(
                    w16[:, 0:Skv_l], v_gath[0, h, b],
                    (((1,), (0,)), ((), ())),
                    preferred_element_type=jnp.float32)
                for c in range(1, N_DEV):
                    ctx = ctx + lax.dot_general(
                        w16[:, c * Skv_l:(c + 1) * Skv_l], v_gath[c, h, b],
                        (((1,), (0,)), ((), ())),
                        preferred_element_type=jnp.float32)
                ctx = ctx / denom
                ctx_buf[b, :, h * DH:(h + 1) * DH] = ctx.astype(bf16)

        po = jnp.stack([
            lax.dot_general(ctx_buf[b], wo_ref[...], (((1,), (0,)), ((), ())),
                            preferred_element_type=jnp.float32)
            for b in range(B)])
        acc_buf[...] = po.astype(bf16)

        lo = 0
        w = Sq
        for r in range(3):
            half = w // 2
            partner = my ^ (1 << r)
            bit = (my >> r) & 1
            keep_lo = lo + bit * half
            send_lo = lo + (1 - bit) * half
            ex = pltpu.make_async_remote_copy(
                src_ref=acc_buf.at[:, pl.ds(send_lo, half), :],
                dst_ref=stage.at[r, :, pl.ds(0, half), :],
                send_sem=send_o.at[1 + r], recv_sem=recv_o.at[1 + r],
                device_id=(partner,), device_id_type=pl.DeviceIdType.MESH)
            ex.start()
            ex.wait()
            acc_buf[:, pl.ds(keep_lo, half), :] = (
                acc_buf[:, pl.ds(keep_lo, half), :]
                + stage[r, :, pl.ds(0, half), :])
            lo = keep_lo
            w = half

        for r in (2, 1, 0):
            partner = my ^ (1 << r)
            bit = (my >> r) & 1
            ex = pltpu.make_async_remote_copy(
                src_ref=acc_buf.at[:, pl.ds(lo, w), :],
                dst_ref=acc_buf.at[:, pl.ds(lo, w), :],
                send_sem=send_o.at[5 + r], recv_sem=recv_o.at[5 + r],
                device_id=(partner,), device_id_type=pl.DeviceIdType.MESH)
            ex.start()
            ex.wait()
            lo = lo - bit * w
            w = w * 2

        for b in range(B):
            out_ref[b] = acc_buf[b].astype(jnp.float32)

        for d in range(1, N_DEV):
            p = (my + d) % N_DEV
            rk, rv = kv_rdma(d, p, my)
            rk.wait_send()
            rv.wait_send()

    return pl.pallas_call(
        body,
        out_shape=jax.ShapeDtypeStruct((B, Sq, Dm), jnp.float32),
        in_specs=[pl.BlockSpec(memory_space=pltpu.VMEM)] * 5,
        out_specs=pl.BlockSpec(memory_space=pltpu.VMEM),
        scratch_shapes=[
            pltpu.VMEM((B, Sq, HL * DH), bf16),
            pltpu.VMEM((N_DEV, HL, B, Skv_l, DH), bf16),
            pltpu.VMEM((N_DEV, HL, B, Skv_l, DH), bf16),
            pltpu.VMEM((Sq, N_DEV * Skv_l), jnp.float32),
            pltpu.VMEM((B, Sq, HL * DH), bf16),
            pltpu.VMEM((B, Sq, Dm), bf16),
            pltpu.VMEM((3, B, Sq // 2, Dm), bf16),
            pltpu.SemaphoreType.DMA((N_DEV,)),
            pltpu.SemaphoreType.DMA((N_DEV,)),
            pltpu.SemaphoreType.DMA((N_DEV,)),
            pltpu.SemaphoreType.DMA((N_DEV,)),
            pltpu.SemaphoreType.DMA((N_DEV,)),
            pltpu.SemaphoreType.DMA((N_DEV,)),
        ],
        compiler_params=pltpu.CompilerParams(collective_id=0),
    )(x16, wq16, k_t, v_t, wo16)
